# baseline (speedup 1.0000x reference)
import functools

import jax
import jax.numpy as jnp
from jax import lax
from jax.experimental import pallas as pl
from jax.experimental.pallas import tpu as pltpu
from jax.experimental.pallas import tpu_sc as plsc

B = 16384
D = 32
H = 64
NU = 1000000
NP = 100000
NI = 1000
NC = 2
NS = 16
NW = NC * NS
L = 16
CH = 12800


def _chunks(n):
    nfull = n // CH
    out = [(k * CH, CH) for k in range(nfull - 1)]
    out.append(((nfull - 1) * CH, CH + n % CH))
    return out


def _sc_gather_body(uid_hbm, pid_hbm, iid_hbm, ut_hbm, pt_hbm, itF_hbm,
                    u_out, p_out, i_out, uscr, pscr,
                    ids_v, col_v,
                    buf0, buf1, buf2, buf3, ubuf_last, pbuf_last,
                    sem_r, sem_w0, sem_w1, sem_g):
    d = lax.axis_index("s") * NC + lax.axis_index("c")
    bufs = (buf0, buf1, buf2, buf3)
    wsems = (sem_w0, sem_w1)

    def detile(tab_hbm, n_rows, scr, last_buf):
        base = d * n_rows
        chunks = _chunks(n_rows)
        n = len(chunks)
        bufmap = [last_buf if k == n - 1 else bufs[k % 4] for k in range(n)]
        reads = [None] * n
        writes = [None] * n
        w_waited = [False] * n

        def issue_read(k):
            off, sz = chunks[k]
            reads[k] = pltpu.async_copy(tab_hbm.at[d, pl.ds(off, sz)],
                                        bufmap[k], sem_r)

        issue_read(0)
        if n > 1:
            issue_read(1)
        for k in range(n):
            if k + 2 < n:
                if k - 2 >= 0 and not w_waited[k - 2]:
                    writes[k - 2].wait()
                    w_waited[k - 2] = True
                issue_read(k + 2)
            reads[k].wait()
            off, sz = chunks[k]
            writes[k] = pltpu.async_copy(bufmap[k],
                                         scr.at[pl.ds(base + off, sz)],
                                         wsems[k % 2])
        for k in range(n):
            if not w_waited[k]:
                writes[k].wait()

    def issue_gather(ids_hbm, n_rows, scr, ids_v, col_v):
        pltpu.sync_copy(ids_hbm, ids_v)
        return pltpu.async_copy(scr.at[pl.ds(d * n_rows, n_rows)].at[ids_v],
                                col_v, sem_g)

    ig = issue_gather(iid_hbm, NI, itF_hbm, ids_v, col_v)
    detile(ut_hbm, NU, uscr, ubuf_last)
    ig.wait()
    pltpu.sync_copy(col_v, i_out.at[d])
    ug = issue_gather(uid_hbm, NU, uscr, ids_v, col_v)
    detile(pt_hbm, NP, pscr, pbuf_last)
    ug.wait()
    pltpu.sync_copy(col_v, u_out.at[d])
    pg = issue_gather(pid_hbm, NP, pscr, ids_v, col_v)
    pg.wait()
    pltpu.sync_copy(col_v, p_out.at[d])


_sc_gather = pl.kernel(
    _sc_gather_body,
    out_type=(
        jax.ShapeDtypeStruct((NW, B), jnp.float32),
        jax.ShapeDtypeStruct((NW, B), jnp.float32),
        jax.ShapeDtypeStruct((NW, B), jnp.float32),
        jax.ShapeDtypeStruct((NW * NU,), jnp.float32),
        jax.ShapeDtypeStruct((NW * NP,), jnp.float32),
    ),
    mesh=plsc.VectorSubcoreMesh(core_axis_name="c", subcore_axis_name="s"),
    scratch_types=[
        pltpu.VMEM((B,), jnp.int32),
        pltpu.VMEM((B,), jnp.float32),
        pltpu.VMEM((CH,), jnp.float32),
        pltpu.VMEM((CH,), jnp.float32),
        pltpu.VMEM((CH,), jnp.float32),
        pltpu.VMEM((CH,), jnp.float32),
        pltpu.VMEM((CH + NU % CH,), jnp.float32),
        pltpu.VMEM((CH + NP % CH,), jnp.float32),
        pltpu.SemaphoreType.DMA,
        pltpu.SemaphoreType.DMA,
        pltpu.SemaphoreType.DMA,
        pltpu.SemaphoreType.DMA,
    ],
    compiler_params=pltpu.CompilerParams(use_tc_tiling_on_sc=True),
)


BS = 2048


def _mlp_body(u_ref, p_ref, i_ref, w1u_ref, w1p_ref, w1i_ref, b1_ref,
              w2_ref, b2_ref, out_ref):
    h = (jnp.dot(w1u_ref[...], u_ref[...], preferred_element_type=jnp.float32)
         + jnp.dot(w1p_ref[...], p_ref[...], preferred_element_type=jnp.float32)
         + jnp.dot(w1i_ref[...], i_ref[...], preferred_element_type=jnp.float32)
         + b1_ref[...])
    h = jnp.maximum(h, 0.0)
    out_ref[...] = (jnp.dot(w2_ref[...], h, preferred_element_type=jnp.float32)
                    + b2_ref[...])


def _tc_mlp(u, p, i, w1uT, w1pT, w1iT, b1c, w2T, b2c):
    grid = (B // BS,)
    emb_spec = pl.BlockSpec((D, BS), lambda j: (0, j))
    full = lambda shape: pl.BlockSpec(shape, lambda j: (0, 0))
    return pl.pallas_call(
        _mlp_body,
        grid=grid,
        in_specs=[emb_spec, emb_spec, emb_spec,
                  full((H, D)), full((H, D)), full((H, D)), full((H, 1)),
                  full((1, H)), full((1, 1))],
        out_specs=pl.BlockSpec((1, BS), lambda j: (0, j)),
        out_shape=jax.ShapeDtypeStruct((1, B), jnp.float32),
    )(u, p, i, w1uT, w1pT, w1iT, b1c, w2T, b2c)


def kernel(user_ids, product_ids, interaction_ids, user_table, product_table,
           interaction_table, W1, b1, W2, b2):
    uid = user_ids.astype(jnp.int32)
    pid = product_ids.astype(jnp.int32)
    iid = interaction_ids.astype(jnp.int32)
    u, p, i, _, _ = _sc_gather(uid, pid, iid, user_table.T, product_table.T,
                               interaction_table.T.reshape(-1))
    w1uT = W1[:D].T
    w1pT = W1[D:2 * D].T
    w1iT = W1[2 * D:].T
    outT = _tc_mlp(u, p, i, w1uT, w1pT, w1iT, b1.reshape(H, 1), W2.T,
                   b2.reshape(1, 1))
    return outT.reshape(B, 1)


# speedup vs baseline: 1.3082x; 1.3082x over previous
import functools

import jax
import jax.numpy as jnp
from jax import lax
from jax.experimental import pallas as pl
from jax.experimental.pallas import tpu as pltpu
from jax.experimental.pallas import tpu_sc as plsc

B = 16384
D = 32
H = 64
NU = 1000000
NP = 100000
NI = 1000
NC = 2
NS = 16
NW = NC * NS
L = 16
CH = 12800


def _chunks(n):
    nfull = n // CH
    out = [(k * CH, CH) for k in range(nfull - 1)]
    out.append(((nfull - 1) * CH, CH + n % CH))
    return out


def _sc_gather_body(uid_hbm, pid_hbm, iid_hbm, ut_hbm, pt_hbm, itF_hbm,
                    u_out, p_out, i_out, uscr, pscr,
                    ids_v, col_v, irow_v,
                    buf0, buf1, buf2, buf3, ubuf_last, pbuf_last,
                    sem_r, sem_w0, sem_w1, sem_g):
    d = lax.axis_index("s") * NC + lax.axis_index("c")
    bufs = (buf0, buf1, buf2, buf3)
    wsems = (sem_w0, sem_w1)

    def detile(tab_hbm, n_rows, scr, last_buf):
        base = d * n_rows
        chunks = _chunks(n_rows)
        n = len(chunks)
        bufmap = [last_buf if k == n - 1 else bufs[k % 4] for k in range(n)]
        reads = [None] * n
        writes = [None] * n
        w_waited = [False] * n

        def issue_read(k):
            off, sz = chunks[k]
            reads[k] = pltpu.async_copy(tab_hbm.at[d, pl.ds(off, sz)],
                                        bufmap[k], sem_r)

        issue_read(0)
        if n > 1:
            issue_read(1)
        for k in range(n):
            if k + 2 < n:
                if k - 2 >= 0 and not w_waited[k - 2]:
                    writes[k - 2].wait()
                    w_waited[k - 2] = True
                issue_read(k + 2)
            reads[k].wait()
            off, sz = chunks[k]
            writes[k] = pltpu.async_copy(bufmap[k],
                                         scr.at[pl.ds(base + off, sz)],
                                         wsems[k % 2])
        for k in range(n):
            if not w_waited[k]:
                writes[k].wait()

    def gather(ids_hbm, n_rows, scr, out_hbm):
        pltpu.sync_copy(ids_hbm, ids_v)
        pltpu.async_copy(scr.at[pl.ds(d * n_rows, n_rows)].at[ids_v],
                         col_v, sem_g).wait()
        pltpu.sync_copy(col_v, out_hbm.at[d])

    pltpu.sync_copy(itF_hbm.at[pl.ds(d * NI, NI)], irow_v)
    pltpu.sync_copy(iid_hbm, ids_v)

    def local_gather(j, _):
        sl = pl.ds(j * L, L)
        col_v[sl] = plsc.load_gather(irow_v, [ids_v[sl]])
        return _

    lax.fori_loop(0, B // L, local_gather, 0, unroll=8)
    pltpu.sync_copy(col_v, i_out.at[d])

    detile(ut_hbm, NU, uscr, ubuf_last)
    gather(uid_hbm, NU, uscr, u_out)
    detile(pt_hbm, NP, pscr, pbuf_last)
    gather(pid_hbm, NP, pscr, p_out)


_sc_gather = pl.kernel(
    _sc_gather_body,
    out_type=(
        jax.ShapeDtypeStruct((NW, B), jnp.float32),
        jax.ShapeDtypeStruct((NW, B), jnp.float32),
        jax.ShapeDtypeStruct((NW, B), jnp.float32),
        jax.ShapeDtypeStruct((NW * NU,), jnp.float32),
        jax.ShapeDtypeStruct((NW * NP,), jnp.float32),
    ),
    mesh=plsc.VectorSubcoreMesh(core_axis_name="c", subcore_axis_name="s"),
    scratch_types=[
        pltpu.VMEM((B,), jnp.int32),
        pltpu.VMEM((B,), jnp.float32),
        pltpu.VMEM((NI,), jnp.float32),
        pltpu.VMEM((CH,), jnp.float32),
        pltpu.VMEM((CH,), jnp.float32),
        pltpu.VMEM((CH,), jnp.float32),
        pltpu.VMEM((CH,), jnp.float32),
        pltpu.VMEM((CH + NU % CH,), jnp.float32),
        pltpu.VMEM((CH + NP % CH,), jnp.float32),
        pltpu.SemaphoreType.DMA,
        pltpu.SemaphoreType.DMA,
        pltpu.SemaphoreType.DMA,
        pltpu.SemaphoreType.DMA,
    ],
    compiler_params=pltpu.CompilerParams(use_tc_tiling_on_sc=True,
                                        needs_layout_passes=False),
)


BS = 2048


def _mlp_body(u_ref, p_ref, i_ref, w1u_ref, w1p_ref, w1i_ref, b1_ref,
              w2_ref, b2_ref, out_ref):
    h = (jnp.dot(w1u_ref[...], u_ref[...], preferred_element_type=jnp.float32)
         + jnp.dot(w1p_ref[...], p_ref[...], preferred_element_type=jnp.float32)
         + jnp.dot(w1i_ref[...], i_ref[...], preferred_element_type=jnp.float32)
         + b1_ref[...])
    h = jnp.maximum(h, 0.0)
    out_ref[...] = (jnp.dot(w2_ref[...], h, preferred_element_type=jnp.float32)
                    + b2_ref[...])


def _tc_mlp(u, p, i, w1uT, w1pT, w1iT, b1c, w2T, b2c):
    grid = (B // BS,)
    emb_spec = pl.BlockSpec((D, BS), lambda j: (0, j))
    full = lambda shape: pl.BlockSpec(shape, lambda j: (0, 0))
    return pl.pallas_call(
        _mlp_body,
        grid=grid,
        in_specs=[emb_spec, emb_spec, emb_spec,
                  full((H, D)), full((H, D)), full((H, D)), full((H, 1)),
                  full((1, H)), full((1, 1))],
        out_specs=pl.BlockSpec((1, BS), lambda j: (0, j)),
        out_shape=jax.ShapeDtypeStruct((1, B), jnp.float32),
    )(u, p, i, w1uT, w1pT, w1iT, b1c, w2T, b2c)


def kernel(user_ids, product_ids, interaction_ids, user_table, product_table,
           interaction_table, W1, b1, W2, b2):
    uid = user_ids.astype(jnp.int32)
    pid = product_ids.astype(jnp.int32)
    iid = interaction_ids.astype(jnp.int32)
    u, p, i, _, _ = _sc_gather(uid, pid, iid, user_table.T, product_table.T,
                               interaction_table.T.reshape(-1))
    w1uT = W1[:D].T
    w1pT = W1[D:2 * D].T
    w1iT = W1[2 * D:].T
    outT = _tc_mlp(u, p, i, w1uT, w1pT, w1iT, b1.reshape(H, 1), W2.T,
                   b2.reshape(1, 1))
    return outT.reshape(B, 1)


# speedup vs baseline: 1.3104x; 1.0017x over previous
import jax
import jax.numpy as jnp
from jax import lax
from jax.experimental import pallas as pl
from jax.experimental.pallas import tpu as pltpu
from jax.experimental.pallas import tpu_sc as plsc

B = 16384
D = 32
H = 64
NU = 1000000
NP = 100000
NI = 1000
NC = 2
NS = 16
NW = NC * NS
L = 16
CH = 12800


def _chunks(n):
    nfull = n // CH
    out = [(k * CH, CH) for k in range(nfull - 1)]
    out.append(((nfull - 1) * CH, CH + n % CH))
    return out


def _sc_gather_body(uid_hbm, pid_hbm, iid_hbm, ut_hbm, pt_hbm, itF_hbm,
                    u_out, p_out, i_out, uscr, pscr,
                    ids_v, col_v, irow_v,
                    buf0, buf1, buf2, buf3, ubuf_last, pbuf_last,
                    sem_r, sem_w0, sem_w1, sem_g):
    d = lax.axis_index("s") * NC + lax.axis_index("c")
    bufs = (buf0, buf1, buf2, buf3)
    wsems = (sem_w0, sem_w1)

    def detile(tab_hbm, n_rows, scr, last_buf):
        base = d * n_rows
        chunks = _chunks(n_rows)
        n = len(chunks)
        bufmap = [last_buf if k == n - 1 else bufs[k % 4] for k in range(n)]
        reads = [None] * n
        writes = [None] * n
        w_waited = [False] * n

        def issue_read(k):
            off, sz = chunks[k]
            reads[k] = pltpu.async_copy(tab_hbm.at[d, pl.ds(off, sz)],
                                        bufmap[k], sem_r)

        issue_read(0)
        if n > 1:
            issue_read(1)
        for k in range(n):
            if k + 2 < n:
                if k - 2 >= 0 and not w_waited[k - 2]:
                    writes[k - 2].wait()
                    w_waited[k - 2] = True
                issue_read(k + 2)
            reads[k].wait()
            off, sz = chunks[k]
            writes[k] = pltpu.async_copy(bufmap[k],
                                         scr.at[pl.ds(base + off, sz)],
                                         wsems[k % 2])
        for k in range(n):
            if not w_waited[k]:
                writes[k].wait()

    def gather(ids_hbm, n_rows, scr, out_hbm):
        pltpu.sync_copy(ids_hbm, ids_v)
        pltpu.async_copy(scr.at[pl.ds(d * n_rows, n_rows)].at[ids_v],
                         col_v, sem_g).wait()
        pltpu.sync_copy(col_v, out_hbm.at[d])

    pltpu.sync_copy(itF_hbm.at[pl.ds(d * NI, NI)], irow_v)
    pltpu.sync_copy(iid_hbm, ids_v)

    def local_gather(j, _):
        sl = pl.ds(j * L, L)
        col_v[sl] = plsc.load_gather(irow_v, [ids_v[sl]])
        return _

    lax.fori_loop(0, B // L, local_gather, 0, unroll=8)
    pltpu.sync_copy(col_v, i_out.at[d])

    detile(ut_hbm, NU, uscr, ubuf_last)
    gather(uid_hbm, NU, uscr, u_out)
    detile(pt_hbm, NP, pscr, pbuf_last)
    gather(pid_hbm, NP, pscr, p_out)


_sc_gather = pl.kernel(
    _sc_gather_body,
    out_type=(
        jax.ShapeDtypeStruct((NW, B), jnp.float32),
        jax.ShapeDtypeStruct((NW, B), jnp.float32),
        jax.ShapeDtypeStruct((NW, B), jnp.float32),
        jax.ShapeDtypeStruct((NW * NU,), jnp.float32),
        jax.ShapeDtypeStruct((NW * NP,), jnp.float32),
    ),
    mesh=plsc.VectorSubcoreMesh(core_axis_name="c", subcore_axis_name="s"),
    scratch_types=[
        pltpu.VMEM((B,), jnp.int32),
        pltpu.VMEM((B,), jnp.float32),
        pltpu.VMEM((NI,), jnp.float32),
        pltpu.VMEM((CH,), jnp.float32),
        pltpu.VMEM((CH,), jnp.float32),
        pltpu.VMEM((CH,), jnp.float32),
        pltpu.VMEM((CH,), jnp.float32),
        pltpu.VMEM((CH + NU % CH,), jnp.float32),
        pltpu.VMEM((CH + NP % CH,), jnp.float32),
        pltpu.SemaphoreType.DMA,
        pltpu.SemaphoreType.DMA,
        pltpu.SemaphoreType.DMA,
        pltpu.SemaphoreType.DMA,
    ],
    compiler_params=pltpu.CompilerParams(use_tc_tiling_on_sc=True,
                                        needs_layout_passes=False),
)


BS = 2048


def _mlp_body(u_ref, p_ref, i_ref, w1u_ref, w1p_ref, w1i_ref, b1_ref,
              w2_ref, b2_ref, out_ref):
    h = (jnp.dot(w1u_ref[...], u_ref[...], preferred_element_type=jnp.float32)
         + jnp.dot(w1p_ref[...], p_ref[...], preferred_element_type=jnp.float32)
         + jnp.dot(w1i_ref[...], i_ref[...], preferred_element_type=jnp.float32)
         + b1_ref[...])
    h = jnp.maximum(h, 0.0)
    out_ref[...] = (jnp.dot(w2_ref[...], h, preferred_element_type=jnp.float32)
                    + b2_ref[...])


def _tc_mlp(u, p, i, w1uT, w1pT, w1iT, b1c, w2T, b2c):
    grid = (B // BS,)
    emb_spec = pl.BlockSpec((D, BS), lambda j: (0, j))
    full = lambda shape: pl.BlockSpec(shape, lambda j: (0, 0))
    return pl.pallas_call(
        _mlp_body,
        grid=grid,
        in_specs=[emb_spec, emb_spec, emb_spec,
                  full((H, D)), full((H, D)), full((H, D)), full((H, 1)),
                  full((1, H)), full((1, 1))],
        out_specs=pl.BlockSpec((1, BS), lambda j: (0, j)),
        out_shape=jax.ShapeDtypeStruct((1, B), jnp.float32),
    )(u, p, i, w1uT, w1pT, w1iT, b1c, w2T, b2c)


def kernel(user_ids, product_ids, interaction_ids, user_table, product_table,
           interaction_table, W1, b1, W2, b2):
    uid = user_ids.astype(jnp.int32)
    pid = product_ids.astype(jnp.int32)
    iid = interaction_ids.astype(jnp.int32)
    u, p, i, _, _ = _sc_gather(uid, pid, iid, user_table.T, product_table.T,
                               interaction_table.T.reshape(-1))
    w1uT = W1[:D].T
    w1pT = W1[D:2 * D].T
    w1iT = W1[2 * D:].T
    outT = _tc_mlp(u, p, i, w1uT, w1pT, w1iT, b1.reshape(H, 1), W2.T,
                   b2.reshape(1, 1))
    return outT.reshape(B, 1)
